# Initial kernel scaffold; baseline (speedup 1.0000x reference)
import jax
import jax.numpy as jnp
from jax.experimental import pallas as pl


def kernel(node_representations, edges, edge_weights, p1_g, p1_b, p1_m, p1_v, p1_W, p1_c, p2_g, p2_b, p2_m, p2_v, p2_W, p2_c, u1_g, u1_b, u1_m, u1_v, u1_W, u1_c, u2_g, u2_b, u2_m, u2_v, u2_W, u2_c):
    raise NotImplementedError("write your pallas kernel here")


# speedup vs baseline: 6.1922x; 6.1922x over previous
import functools

import jax
import jax.numpy as jnp
from jax import lax
from jax.experimental import pallas as pl
from jax.experimental.pallas import tpu as pltpu
from jax.experimental.pallas import tpu_sc as plsc

_NC = 2
_NS = 16
_L = 16
_NW = _NC * _NS
_CH = 80
_ZB = 640


def _gelu(x):
    return x * 0.5 * (1.0 + lax.erf(x * 0.7071067811865476))


def _bn(x, g, b, m, v):
    s = g * lax.rsqrt(v + 1e-3)
    return x * s + (b - m * s)


def _prep_body(x_ref, g1, b1, m1, v1, W1, c1, g2, b2, m2, v2, W2, c2, o_ref):
    h = _bn(x_ref[...], g1[...], b1[...], m1[...], v1[...])
    h = _gelu(jnp.dot(h, W1[...], preferred_element_type=jnp.float32) + c1[...])
    h = _bn(h, g2[...], b2[...], m2[...], v2[...])
    o_ref[...] = _gelu(jnp.dot(h, W2[...], preferred_element_type=jnp.float32) + c2[...])


def _prep_call(x, p1, p2, rb):
    n, d = x.shape
    h = p1[4].shape[1]
    grid = (n // rb,)

    def full(shape):
        return pl.BlockSpec(shape, lambda i: (0, 0))

    in_specs = [pl.BlockSpec((rb, d), lambda i: (i, 0))]
    for (g, b, m, v, W, c) in (p1, p2):
        din, dout = W.shape
        in_specs += [full((1, din))] * 4 + [full((din, dout)), full((1, dout))]
    args = [x]
    for (g, b, m, v, W, c) in (p1, p2):
        args += [g.reshape(1, -1), b.reshape(1, -1), m.reshape(1, -1),
                 v.reshape(1, -1), W, c.reshape(1, -1)]
    return pl.pallas_call(
        _prep_body,
        grid=grid,
        in_specs=in_specs,
        out_specs=pl.BlockSpec((rb, h), lambda i: (i, 0)),
        out_shape=jax.ShapeDtypeStruct((n, h), jnp.float32),
    )(*args)


def _sc_aggregate(msg, src, dst, w):
    n, h = msg.shape
    e = src.shape[0]
    epw = e // _NW
    nchunk = epw // _CH
    assert epw * _NW == e and nchunk * _CH == epw and h % _L == 0

    mesh = plsc.VectorSubcoreMesh(core_axis_name="c", subcore_axis_name="s")

    @functools.partial(
        pl.kernel,
        out_type=(jax.ShapeDtypeStruct((_NC * n, h), jnp.float32),
                  jax.ShapeDtypeStruct((_NC * n,), jnp.float32)),
        mesh=mesh,
        scratch_types=[
            pltpu.VMEM((_CH,), jnp.int32),
            pltpu.VMEM((_CH,), jnp.int32),
            pltpu.VMEM((_CH,), jnp.float32),
            pltpu.VMEM((_CH, h), jnp.float32),
            pltpu.VMEM((_CH,), jnp.float32),
            pltpu.VMEM_SHARED((n, h), jnp.float32),
            pltpu.VMEM_SHARED((n,), jnp.float32),
            pltpu.SemaphoreType.DMA,
        ],
        compiler_params=pltpu.CompilerParams(use_tc_tiling_on_sc=False),
    )
    def body(msg_hbm, src_hbm, dst_hbm, w_hbm, sums_out, cnt_out,
             sidx, didx, wv, rows, onesb, acc, accc, sem):
        c = lax.axis_index("c")
        s = lax.axis_index("s")
        wid = s * _NC + c

        for j in range(_CH // _L):
            onesb[pl.ds(j * _L, _L)] = jnp.zeros((_L,), jnp.float32)

        def zrow(i, carry):
            for j in range(h // _L):
                rows[i, pl.ds(j * _L, _L)] = jnp.zeros((_L,), jnp.float32)
            return carry
        lax.fori_loop(0, _CH, zrow, 0)

        r0 = s * _ZB

        def zacc(j, carry):
            off = r0 + j * _CH

            @pl.when(off < n)
            def _():
                pltpu.sync_copy(rows, acc.at[pl.ds(off, _CH)])
                pltpu.sync_copy(onesb, accc.at[pl.ds(off, _CH)])
            return carry
        lax.fori_loop(0, _ZB // _CH, zacc, 0)

        plsc.subcore_barrier()

        for j in range(_CH // _L):
            onesb[pl.ds(j * _L, _L)] = jnp.ones((_L,), jnp.float32)

        ebase = wid * epw

        def chunk(k, carry):
            base = ebase + k * _CH
            pltpu.sync_copy(src_hbm.at[pl.ds(base, _CH)], sidx)
            pltpu.sync_copy(dst_hbm.at[pl.ds(base, _CH)], didx)
            pltpu.sync_copy(w_hbm.at[pl.ds(base, _CH)], wv)
            pltpu.async_copy(msg_hbm.at[sidx], rows, sem).wait()

            def scale(g, cc):
                wvec = wv[pl.ds(g * _L, _L)]
                for j in range(_L):
                    i = g * _L + j
                    wi = wvec[j]
                    for q in range(h // _L):
                        rows[i, pl.ds(q * _L, _L)] = rows[i, pl.ds(q * _L, _L)] * wi
                return cc
            lax.fori_loop(0, _CH // _L, scale, 0)

            pltpu.sync_copy(rows, acc.at[didx], add=True)
            pltpu.sync_copy(onesb, accc.at[didx], add=True)
            return carry
        lax.fori_loop(0, nchunk, chunk, 0)

        plsc.subcore_barrier()

        obase = c * n

        def cpout(j, carry):
            off = r0 + j * _CH

            @pl.when(off < n)
            def _():
                pltpu.sync_copy(acc.at[pl.ds(off, _CH)], rows)
                pltpu.sync_copy(rows, sums_out.at[pl.ds(obase + off, _CH)])
                pltpu.sync_copy(accc.at[pl.ds(off, _CH)], onesb)
                pltpu.sync_copy(onesb, cnt_out.at[pl.ds(obase + off, _CH)])
            return carry
        lax.fori_loop(0, _ZB // _CH, cpout, 0)

    return body(msg, src, dst, w)


def _upd_body(x_ref, sp_ref, cp_ref,
              g3x, b3x, m3x, v3x, g3a, b3a, m3a, v3a, W3x, W3a, c3,
              g4, b4, m4, v4, W4, c4, o_ref):
    sums = sp_ref[0] + sp_ref[1]
    cnt = cp_ref[0] + cp_ref[1]
    agg = sums / jnp.maximum(cnt, 1.0)
    hx = _bn(x_ref[...], g3x[...], b3x[...], m3x[...], v3x[...])
    ha = _bn(agg, g3a[...], b3a[...], m3a[...], v3a[...])
    t = _gelu(jnp.dot(hx, W3x[...], preferred_element_type=jnp.float32)
              + jnp.dot(ha, W3a[...], preferred_element_type=jnp.float32)
              + c3[...])
    t = _bn(t, g4[...], b4[...], m4[...], v4[...])
    o_ref[...] = _gelu(jnp.dot(t, W4[...], preferred_element_type=jnp.float32) + c4[...])


def _upd_call(x, sums_p, cnt_p, u1, u2, rb):
    n, d = x.shape
    h = sums_p.shape[2]
    grid = (n // rb,)
    g3, b3, m3, v3, W3, c3 = u1

    def full(shape):
        return pl.BlockSpec(shape, lambda i: tuple(0 for _ in shape))

    in_specs = [
        pl.BlockSpec((rb, d), lambda i: (i, 0)),
        pl.BlockSpec((_NC, rb, h), lambda i: (0, i, 0)),
        pl.BlockSpec((_NC, rb, 1), lambda i: (0, i, 0)),
    ]
    in_specs += [full((1, d))] * 4 + [full((1, h))] * 4
    in_specs += [full((d, h)), full((h, h)), full((1, h))]
    in_specs += [full((1, h))] * 4 + [full((h, h)), full((1, h))]

    args = [x, sums_p, cnt_p,
            g3[:d].reshape(1, d), b3[:d].reshape(1, d),
            m3[:d].reshape(1, d), v3[:d].reshape(1, d),
            g3[d:].reshape(1, h), b3[d:].reshape(1, h),
            m3[d:].reshape(1, h), v3[d:].reshape(1, h),
            W3[:d], W3[d:], c3.reshape(1, h)]
    g4, b4, m4, v4, W4, c4 = u2
    args += [g4.reshape(1, h), b4.reshape(1, h), m4.reshape(1, h),
             v4.reshape(1, h), W4, c4.reshape(1, h)]
    return pl.pallas_call(
        _upd_body,
        grid=grid,
        in_specs=in_specs,
        out_specs=pl.BlockSpec((rb, h), lambda i: (i, 0)),
        out_shape=jax.ShapeDtypeStruct((n, h), jnp.float32),
    )(*args)


def kernel(node_representations, edges, edge_weights,
           p1_g, p1_b, p1_m, p1_v, p1_W, p1_c,
           p2_g, p2_b, p2_m, p2_v, p2_W, p2_c,
           u1_g, u1_b, u1_m, u1_v, u1_W, u1_c,
           u2_g, u2_b, u2_m, u2_v, u2_W, u2_c):
    x = node_representations.astype(jnp.float32)
    n, d = x.shape
    src = edges[1].astype(jnp.int32)
    dst = edges[0].astype(jnp.int32)
    w = edge_weights.astype(jnp.float32)

    msg = _prep_call(x, (p1_g, p1_b, p1_m, p1_v, p1_W, p1_c),
                     (p2_g, p2_b, p2_m, p2_v, p2_W, p2_c), rb=2000)
    h = msg.shape[1]
    sums2, cnt2 = _sc_aggregate(msg, src, dst, w)
    sums_p = sums2.reshape(_NC, n, h)
    cnt_p = cnt2.reshape(_NC, n, 1)
    return _upd_call(x, sums_p, cnt_p,
                     (u1_g, u1_b, u1_m, u1_v, u1_W, u1_c),
                     (u2_g, u2_b, u2_m, u2_v, u2_W, u2_c), rb=2000)


# speedup vs baseline: 18.1596x; 2.9327x over previous
import functools

import jax
import jax.numpy as jnp
from jax import lax
from jax.experimental import pallas as pl
from jax.experimental.pallas import tpu as pltpu
from jax.experimental.pallas import tpu_sc as plsc

_NC = 2
_NS = 16
_L = 16
_NW = _NC * _NS
_CH = 80
_ZB = 640


def _gelu(x):
    return x * 0.5 * (1.0 + lax.erf(x * 0.7071067811865476))


def _bn(x, g, b, m, v):
    s = g * lax.rsqrt(v + 1e-3)
    return x * s + (b - m * s)


def _prep_body(x_ref, g1, b1, m1, v1, W1, c1, g2, b2, m2, v2, W2, c2, o_ref):
    h = _bn(x_ref[...], g1[...], b1[...], m1[...], v1[...])
    h = _gelu(jnp.dot(h, W1[...], preferred_element_type=jnp.float32) + c1[...])
    h = _bn(h, g2[...], b2[...], m2[...], v2[...])
    o_ref[...] = _gelu(jnp.dot(h, W2[...], preferred_element_type=jnp.float32) + c2[...])


def _prep_call(x, p1, p2, rb):
    n, d = x.shape
    h = p1[4].shape[1]
    grid = (n // rb,)

    def full(shape):
        return pl.BlockSpec(shape, lambda i: (0, 0))

    in_specs = [pl.BlockSpec((rb, d), lambda i: (i, 0))]
    for (g, b, m, v, W, c) in (p1, p2):
        din, dout = W.shape
        in_specs += [full((1, din))] * 4 + [full((din, dout)), full((1, dout))]
    args = [x]
    for (g, b, m, v, W, c) in (p1, p2):
        args += [g.reshape(1, -1), b.reshape(1, -1), m.reshape(1, -1),
                 v.reshape(1, -1), W, c.reshape(1, -1)]
    return pl.pallas_call(
        _prep_body,
        grid=grid,
        in_specs=in_specs,
        out_specs=pl.BlockSpec((rb, h), lambda i: (i, 0)),
        out_shape=jax.ShapeDtypeStruct((n, h), jnp.float32),
    )(*args)


_NB = 4
_LA = 2


def _sc_aggregate(msg, src2, dst2, w):
    n, h = msg.shape
    nw, nchunk, ch = src2.shape
    epw = nchunk * ch
    assert nw == _NW and ch == _CH and h % _L == 0

    mesh = plsc.VectorSubcoreMesh(core_axis_name="c", subcore_axis_name="s")

    @functools.partial(
        pl.kernel,
        out_type=(jax.ShapeDtypeStruct((_NC * n, h), jnp.float32),
                  jax.ShapeDtypeStruct((_NC * n,), jnp.float32)),
        mesh=mesh,
        scratch_types=[
            pltpu.VMEM((nchunk, _CH), jnp.int32),
            pltpu.VMEM((nchunk, _CH), jnp.int32),
            pltpu.VMEM((epw,), jnp.float32),
            pltpu.VMEM((_NB, _CH, h), jnp.float32),
            pltpu.VMEM((_CH,), jnp.float32),
            pltpu.VMEM_SHARED((n, h), jnp.float32),
            pltpu.VMEM_SHARED((n,), jnp.float32),
            pltpu.SemaphoreType.DMA((_NB,)),
            pltpu.SemaphoreType.DMA((_NB,)),
        ],
        compiler_params=pltpu.CompilerParams(use_tc_tiling_on_sc=False,
                                             needs_layout_passes=False),
    )
    def body(msg_hbm, src_hbm, dst_hbm, w_hbm, sums_out, cnt_out,
             sidx, didx, wv, ring, onesb, acc, accc, gsem, ssem):
        c = lax.axis_index("c")
        s = lax.axis_index("s")
        wid = s * _NC + c

        pltpu.sync_copy(src_hbm.at[wid], sidx)
        pltpu.sync_copy(dst_hbm.at[wid], didx)
        pltpu.sync_copy(w_hbm.at[pl.ds(wid * epw, epw)], wv)

        def wchk(i, acc0):
            v = wv[pl.ds(i * _L, _L)]
            nbad = plsc.all_reduce_population_count(v != 1.0)
            return acc0 + nbad[0]
        wdiff = lax.fori_loop(0, epw // _L, wchk, jnp.int32(0))
        allones = wdiff == 0

        for j in range(_CH // _L):
            onesb[pl.ds(j * _L, _L)] = jnp.zeros((_L,), jnp.float32)

        def zrow(i, carry):
            for j in range(h // _L):
                ring[0, i, pl.ds(j * _L, _L)] = jnp.zeros((_L,), jnp.float32)
            return carry
        lax.fori_loop(0, _CH, zrow, 0)

        r0 = s * _ZB
        zb = ring.at[0]
        for j in range(_ZB // _CH):
            off = r0 + j * _CH

            @pl.when(off < n)
            def _():
                pltpu.async_copy(zb, acc.at[pl.ds(off, _CH)], gsem.at[0])
                pltpu.async_copy(onesb, accc.at[pl.ds(off, _CH)], gsem.at[1])
        for j in range(_ZB // _CH):
            off = r0 + j * _CH

            @pl.when(off < n)
            def _():
                pltpu.make_async_copy(zb, acc.at[pl.ds(off, _CH)], gsem.at[0]).wait()
                pltpu.make_async_copy(onesb, accc.at[pl.ds(off, _CH)], gsem.at[1]).wait()

        plsc.subcore_barrier()

        for j in range(_CH // _L):
            onesb[pl.ds(j * _L, _L)] = jnp.ones((_L,), jnp.float32)

        def fire_gather(k, b):
            pltpu.async_copy(msg_hbm.at[sidx.at[k]], ring.at[b], gsem.at[b])

        def drain_gather(b):
            pltpu.make_async_copy(msg_hbm.at[pl.ds(0, _CH)], ring.at[b],
                                  gsem.at[b]).wait()

        def fire_scatter(k, b):
            pltpu.async_copy(ring.at[b], acc.at[didx.at[k]], ssem.at[b],
                             add=True)
            pltpu.async_copy(onesb, accc.at[didx.at[k]], ssem.at[b], add=True)

        def drain_scatter(b):
            pltpu.make_async_copy(msg_hbm.at[pl.ds(0, _CH)], ring.at[b],
                                  ssem.at[b]).wait()
            pltpu.make_async_copy(w_hbm.at[pl.ds(0, _CH)], onesb,
                                  ssem.at[b]).wait()

        def scale(k, b):
            @pl.when(jnp.logical_not(allones))
            def _():
                def sgrp(g, cc):
                    wvec = wv[pl.ds(k * _CH + g * _L, _L)]
                    for j in range(_L):
                        wi = wvec[j]
                        i = g * _L + j
                        for q in range(h // _L):
                            ring[b, i, pl.ds(q * _L, _L)] = (
                                ring[b, i, pl.ds(q * _L, _L)] * wi)
                    return cc
                lax.fori_loop(0, _CH // _L, sgrp, 0)

        def step(k, b):
            kp = k + _LA
            bp = (b + _LA) % _NB

            @pl.when(kp < nchunk)
            def _():
                @pl.when(k >= _LA)
                def _():
                    drain_scatter(bp)
                fire_gather(kp, bp)
            drain_gather(b)
            scale(k, b)
            fire_scatter(k, b)

        for b in range(_LA):
            fire_gather(jnp.int32(b), b)

        nmain = (nchunk // _NB) * _NB

        def ring_step(g, carry):
            for b in range(_NB):
                step(g * _NB + b, b)
            return carry
        lax.fori_loop(0, nmain // _NB, ring_step, 0)

        for k in range(nmain, nchunk):
            step(jnp.int32(k), k % _NB)
        for k in range(nchunk - _NB, nchunk):
            drain_scatter(k % _NB)

        plsc.subcore_barrier()

        obase = c * n
        for j in range(_ZB // _CH):
            off = r0 + j * _CH
            b = j % _NB

            @pl.when(off < n)
            def _():
                if j >= _NB:
                    pltpu.make_async_copy(
                        ring.at[b],
                        sums_out.at[pl.ds(obase + off - _NB * _CH, _CH)],
                        gsem.at[b]).wait()
                pltpu.sync_copy(acc.at[pl.ds(off, _CH)], ring.at[b])
                pltpu.async_copy(ring.at[b], sums_out.at[pl.ds(obase + off, _CH)],
                                 gsem.at[b])
                pltpu.sync_copy(accc.at[pl.ds(off, _CH)], onesb)
                pltpu.sync_copy(onesb, cnt_out.at[pl.ds(obase + off, _CH)])
        for j in range(_ZB // _CH):
            off = r0 + j * _CH
            b = j % _NB
            fired = off < n
            if j + _NB < _ZB // _CH:
                drained = (r0 + (j + _NB) * _CH) < n
                cond = jnp.logical_and(fired, jnp.logical_not(drained))
            else:
                cond = fired

            @pl.when(cond)
            def _():
                pltpu.make_async_copy(
                    ring.at[b], sums_out.at[pl.ds(obase + off, _CH)],
                    gsem.at[b]).wait()

    return body(msg, src2, dst2, w)


def _upd_body(x_ref, sp_ref, cp_ref,
              g3x, b3x, m3x, v3x, g3a, b3a, m3a, v3a, W3x, W3a, c3,
              g4, b4, m4, v4, W4, c4, o_ref):
    sums = sp_ref[0] + sp_ref[1]
    cnt = cp_ref[0] + cp_ref[1]
    agg = sums / jnp.maximum(cnt, 1.0)
    hx = _bn(x_ref[...], g3x[...], b3x[...], m3x[...], v3x[...])
    ha = _bn(agg, g3a[...], b3a[...], m3a[...], v3a[...])
    t = _gelu(jnp.dot(hx, W3x[...], preferred_element_type=jnp.float32)
              + jnp.dot(ha, W3a[...], preferred_element_type=jnp.float32)
              + c3[...])
    t = _bn(t, g4[...], b4[...], m4[...], v4[...])
    o_ref[...] = _gelu(jnp.dot(t, W4[...], preferred_element_type=jnp.float32) + c4[...])


def _upd_call(x, sums_p, cnt_p, u1, u2, rb):
    n, d = x.shape
    h = sums_p.shape[2]
    grid = (n // rb,)
    g3, b3, m3, v3, W3, c3 = u1

    def full(shape):
        return pl.BlockSpec(shape, lambda i: tuple(0 for _ in shape))

    in_specs = [
        pl.BlockSpec((rb, d), lambda i: (i, 0)),
        pl.BlockSpec((_NC, rb, h), lambda i: (0, i, 0)),
        pl.BlockSpec((_NC, rb, 1), lambda i: (0, i, 0)),
    ]
    in_specs += [full((1, d))] * 4 + [full((1, h))] * 4
    in_specs += [full((d, h)), full((h, h)), full((1, h))]
    in_specs += [full((1, h))] * 4 + [full((h, h)), full((1, h))]

    args = [x, sums_p, cnt_p,
            g3[:d].reshape(1, d), b3[:d].reshape(1, d),
            m3[:d].reshape(1, d), v3[:d].reshape(1, d),
            g3[d:].reshape(1, h), b3[d:].reshape(1, h),
            m3[d:].reshape(1, h), v3[d:].reshape(1, h),
            W3[:d], W3[d:], c3.reshape(1, h)]
    g4, b4, m4, v4, W4, c4 = u2
    args += [g4.reshape(1, h), b4.reshape(1, h), m4.reshape(1, h),
             v4.reshape(1, h), W4, c4.reshape(1, h)]
    return pl.pallas_call(
        _upd_body,
        grid=grid,
        in_specs=in_specs,
        out_specs=pl.BlockSpec((rb, h), lambda i: (i, 0)),
        out_shape=jax.ShapeDtypeStruct((n, h), jnp.float32),
    )(*args)


def kernel(node_representations, edges, edge_weights,
           p1_g, p1_b, p1_m, p1_v, p1_W, p1_c,
           p2_g, p2_b, p2_m, p2_v, p2_W, p2_c,
           u1_g, u1_b, u1_m, u1_v, u1_W, u1_c,
           u2_g, u2_b, u2_m, u2_v, u2_W, u2_c):
    x = node_representations.astype(jnp.float32)
    n, d = x.shape
    e = edges.shape[1]
    nchunk = e // (_NW * _CH)
    src2 = edges[1].astype(jnp.int32).reshape(_NW, nchunk, _CH)
    dst2 = edges[0].astype(jnp.int32).reshape(_NW, nchunk, _CH)
    w = edge_weights.astype(jnp.float32)

    msg = _prep_call(x, (p1_g, p1_b, p1_m, p1_v, p1_W, p1_c),
                     (p2_g, p2_b, p2_m, p2_v, p2_W, p2_c), rb=2000)
    h = msg.shape[1]
    sums2, cnt2 = _sc_aggregate(msg, src2, dst2, w)
    sums_p = sums2.reshape(_NC, n, h)
    cnt_p = cnt2.reshape(_NC, n, 1)
    return _upd_call(x, sums_p, cnt_p,
                     (u1_g, u1_b, u1_m, u1_v, u1_W, u1_c),
                     (u2_g, u2_b, u2_m, u2_v, u2_W, u2_c), rb=2000)


# speedup vs baseline: 19.3861x; 1.0675x over previous
import functools

import jax
import jax.numpy as jnp
from jax import lax
from jax.experimental import pallas as pl
from jax.experimental.pallas import tpu as pltpu
from jax.experimental.pallas import tpu_sc as plsc

_NC = 2
_NS = 16
_L = 16
_NW = _NC * _NS
_CH = 80
_ZB = 640
_HP = 48


def _gelu(x):
    return x * 0.5 * (1.0 + lax.erf(x * 0.7071067811865476))


def _bn(x, g, b, m, v):
    s = g * lax.rsqrt(v + 1e-3)
    return x * s + (b - m * s)


def _prep_body(x_ref, g1, b1, m1, v1, W1, c1, g2, b2, m2, v2, W2, c2, o_ref):
    h = _bn(x_ref[...], g1[...], b1[...], m1[...], v1[...])
    h = _gelu(jnp.dot(h, W1[...], preferred_element_type=jnp.float32) + c1[...])
    h = _bn(h, g2[...], b2[...], m2[...], v2[...])
    m = _gelu(jnp.dot(h, W2[...], preferred_element_type=jnp.float32) + c2[...])
    o_ref[...] = jnp.concatenate(
        [m, jnp.ones((m.shape[0], _HP - m.shape[1]), jnp.float32)], axis=1)


def _prep_call(x, p1, p2, rb):
    n, d = x.shape
    h = p1[4].shape[1]
    grid = (n // rb,)

    def full(shape):
        return pl.BlockSpec(shape, lambda i: (0, 0))

    in_specs = [pl.BlockSpec((rb, d), lambda i: (i, 0))]
    for (g, b, m, v, W, c) in (p1, p2):
        din, dout = W.shape
        in_specs += [full((1, din))] * 4 + [full((din, dout)), full((1, dout))]
    args = [x]
    for (g, b, m, v, W, c) in (p1, p2):
        args += [g.reshape(1, -1), b.reshape(1, -1), m.reshape(1, -1),
                 v.reshape(1, -1), W, c.reshape(1, -1)]
    return pl.pallas_call(
        _prep_body,
        grid=grid,
        in_specs=in_specs,
        out_specs=pl.BlockSpec((rb, _HP), lambda i: (i, 0)),
        out_shape=jax.ShapeDtypeStruct((n, _HP), jnp.float32),
    )(*args)


_NB = 4
_LA = 2


def _sc_aggregate(msg, src2, dst2, w):
    n, h = msg.shape
    nw, nchunk, ch = src2.shape
    epw = nchunk * ch
    assert nw == _NW and ch == _CH and h % _L == 0

    mesh = plsc.VectorSubcoreMesh(core_axis_name="c", subcore_axis_name="s")

    @functools.partial(
        pl.kernel,
        out_type=jax.ShapeDtypeStruct((_NC * n, h), jnp.float32),
        mesh=mesh,
        scratch_types=[
            pltpu.VMEM((nchunk, _CH), jnp.int32),
            pltpu.VMEM((nchunk, _CH), jnp.int32),
            pltpu.VMEM((epw,), jnp.float32),
            pltpu.VMEM((_NB, _CH, h), jnp.float32),
            pltpu.VMEM_SHARED((n, h), jnp.float32),
            pltpu.SemaphoreType.DMA((_NB,)),
            pltpu.SemaphoreType.DMA((_NB,)),
        ],
        compiler_params=pltpu.CompilerParams(use_tc_tiling_on_sc=False,
                                             needs_layout_passes=False),
    )
    def body(msg_hbm, src_hbm, dst_hbm, w_hbm, sums_out,
             sidx, didx, wv, ring, acc, gsem, ssem):
        c = lax.axis_index("c")
        s = lax.axis_index("s")
        wid = s * _NC + c

        pltpu.sync_copy(src_hbm.at[wid], sidx)
        pltpu.sync_copy(dst_hbm.at[wid], didx)
        pltpu.sync_copy(w_hbm.at[pl.ds(wid * epw, epw)], wv)

        def wchk(i, acc0):
            v = wv[pl.ds(i * _L, _L)]
            nbad = plsc.all_reduce_population_count(v != 1.0)
            return acc0 + nbad[0]
        wdiff = lax.fori_loop(0, epw // _L, wchk, jnp.int32(0))
        allones = wdiff == 0

        def zrow(i, carry):
            for j in range(h // _L):
                ring[0, i, pl.ds(j * _L, _L)] = jnp.zeros((_L,), jnp.float32)
            return carry
        lax.fori_loop(0, _CH, zrow, 0)

        r0 = s * _ZB
        zb = ring.at[0]
        for j in range(_ZB // _CH):
            off = r0 + j * _CH

            @pl.when(off < n)
            def _():
                pltpu.async_copy(zb, acc.at[pl.ds(off, _CH)], gsem.at[0])
        for j in range(_ZB // _CH):
            off = r0 + j * _CH

            @pl.when(off < n)
            def _():
                pltpu.make_async_copy(zb, acc.at[pl.ds(off, _CH)], gsem.at[0]).wait()

        plsc.subcore_barrier()

        def fire_gather(k, b):
            pltpu.async_copy(msg_hbm.at[sidx.at[k]], ring.at[b], gsem.at[b])

        def drain_gather(b):
            pltpu.make_async_copy(msg_hbm.at[pl.ds(0, _CH)], ring.at[b],
                                  gsem.at[b]).wait()

        def fire_scatter(k, b):
            pltpu.async_copy(ring.at[b], acc.at[didx.at[k]], ssem.at[b],
                             add=True)

        def drain_scatter(b):
            pltpu.make_async_copy(msg_hbm.at[pl.ds(0, _CH)], ring.at[b],
                                  ssem.at[b]).wait()

        def scale(k, b):
            @pl.when(jnp.logical_not(allones))
            def _():
                def sgrp(g, cc):
                    wvec = wv[pl.ds(k * _CH + g * _L, _L)]
                    for j in range(_L):
                        wi = wvec[j]
                        i = g * _L + j
                        for q in range((h - _L) // _L):
                            ring[b, i, pl.ds(q * _L, _L)] = (
                                ring[b, i, pl.ds(q * _L, _L)] * wi)
                    return cc
                lax.fori_loop(0, _CH // _L, sgrp, 0)

        def step(k, b):
            kp = k + _LA
            bp = (b + _LA) % _NB

            @pl.when(kp < nchunk)
            def _():
                @pl.when(k >= _LA)
                def _():
                    drain_scatter(bp)
                fire_gather(kp, bp)
            drain_gather(b)
            scale(k, b)
            fire_scatter(k, b)

        for b in range(_LA):
            fire_gather(jnp.int32(b), b)

        nmain = (nchunk // _NB) * _NB

        def ring_step(g, carry):
            for b in range(_NB):
                step(g * _NB + b, b)
            return carry
        lax.fori_loop(0, nmain // _NB, ring_step, 0)

        for k in range(nmain, nchunk):
            step(jnp.int32(k), k % _NB)
        for k in range(nchunk - _NB, nchunk):
            drain_scatter(k % _NB)

        plsc.subcore_barrier()

        obase = c * n
        for j in range(_ZB // _CH):
            off = r0 + j * _CH
            b = j % _NB

            @pl.when(off < n)
            def _():
                if j >= _NB:
                    pltpu.make_async_copy(
                        ring.at[b],
                        sums_out.at[pl.ds(obase + off - _NB * _CH, _CH)],
                        gsem.at[b]).wait()
                pltpu.sync_copy(acc.at[pl.ds(off, _CH)], ring.at[b])
                pltpu.async_copy(ring.at[b], sums_out.at[pl.ds(obase + off, _CH)],
                                 gsem.at[b])
        for j in range(_ZB // _CH):
            off = r0 + j * _CH
            b = j % _NB
            fired = off < n
            if j + _NB < _ZB // _CH:
                drained = (r0 + (j + _NB) * _CH) < n
                cond = jnp.logical_and(fired, jnp.logical_not(drained))
            else:
                cond = fired

            @pl.when(cond)
            def _():
                pltpu.make_async_copy(
                    ring.at[b], sums_out.at[pl.ds(obase + off, _CH)],
                    gsem.at[b]).wait()

    return body(msg, src2, dst2, w)


def _upd_body(x_ref, sp_ref,
              g3x, b3x, m3x, v3x, g3a, b3a, m3a, v3a, W3x, W3a, c3,
              g4, b4, m4, v4, W4, c4, o_ref):
    s48 = sp_ref[0] + sp_ref[1]
    sums = s48[:, : s48.shape[1] - _L]
    cnt = s48[:, s48.shape[1] - _L: s48.shape[1] - _L + 1]
    agg = sums / jnp.maximum(cnt, 1.0)
    hx = _bn(x_ref[...], g3x[...], b3x[...], m3x[...], v3x[...])
    ha = _bn(agg, g3a[...], b3a[...], m3a[...], v3a[...])
    t = _gelu(jnp.dot(hx, W3x[...], preferred_element_type=jnp.float32)
              + jnp.dot(ha, W3a[...], preferred_element_type=jnp.float32)
              + c3[...])
    t = _bn(t, g4[...], b4[...], m4[...], v4[...])
    o_ref[...] = _gelu(jnp.dot(t, W4[...], preferred_element_type=jnp.float32) + c4[...])


def _upd_call(x, sums_p, u1, u2, rb):
    n, d = x.shape
    h = sums_p.shape[2] - _L
    grid = (n // rb,)
    g3, b3, m3, v3, W3, c3 = u1

    def full(shape):
        return pl.BlockSpec(shape, lambda i: tuple(0 for _ in shape))

    in_specs = [
        pl.BlockSpec((rb, d), lambda i: (i, 0)),
        pl.BlockSpec((_NC, rb, _HP), lambda i: (0, i, 0)),
    ]
    in_specs += [full((1, d))] * 4 + [full((1, h))] * 4
    in_specs += [full((d, h)), full((h, h)), full((1, h))]
    in_specs += [full((1, h))] * 4 + [full((h, h)), full((1, h))]

    args = [x, sums_p,
            g3[:d].reshape(1, d), b3[:d].reshape(1, d),
            m3[:d].reshape(1, d), v3[:d].reshape(1, d),
            g3[d:].reshape(1, h), b3[d:].reshape(1, h),
            m3[d:].reshape(1, h), v3[d:].reshape(1, h),
            W3[:d], W3[d:], c3.reshape(1, h)]
    g4, b4, m4, v4, W4, c4 = u2
    args += [g4.reshape(1, h), b4.reshape(1, h), m4.reshape(1, h),
             v4.reshape(1, h), W4, c4.reshape(1, h)]
    return pl.pallas_call(
        _upd_body,
        grid=grid,
        in_specs=in_specs,
        out_specs=pl.BlockSpec((rb, h), lambda i: (i, 0)),
        out_shape=jax.ShapeDtypeStruct((n, h), jnp.float32),
    )(*args)


def kernel(node_representations, edges, edge_weights,
           p1_g, p1_b, p1_m, p1_v, p1_W, p1_c,
           p2_g, p2_b, p2_m, p2_v, p2_W, p2_c,
           u1_g, u1_b, u1_m, u1_v, u1_W, u1_c,
           u2_g, u2_b, u2_m, u2_v, u2_W, u2_c):
    x = node_representations.astype(jnp.float32)
    n, d = x.shape
    e = edges.shape[1]
    nchunk = e // (_NW * _CH)
    src2 = edges[1].astype(jnp.int32).reshape(_NW, nchunk, _CH)
    dst2 = edges[0].astype(jnp.int32).reshape(_NW, nchunk, _CH)
    w = edge_weights.astype(jnp.float32)

    msg = _prep_call(x, (p1_g, p1_b, p1_m, p1_v, p1_W, p1_c),
                     (p2_g, p2_b, p2_m, p2_v, p2_W, p2_c), rb=2000)
    sums2 = _sc_aggregate(msg, src2, dst2, w)
    sums_p = sums2.reshape(_NC, n, _HP)
    return _upd_call(x, sums_p,
                     (u1_g, u1_b, u1_m, u1_v, u1_W, u1_c),
                     (u2_g, u2_b, u2_m, u2_v, u2_W, u2_c), rb=2000)


# speedup vs baseline: 23.2170x; 1.1976x over previous
import functools

import jax
import jax.numpy as jnp
from jax import lax
from jax.experimental import pallas as pl
from jax.experimental.pallas import tpu as pltpu
from jax.experimental.pallas import tpu_sc as plsc

_NC = 2
_NS = 16
_L = 16
_NW = _NC * _NS
_CH = 80
_ZB = 640
_HP = 48


def _gelu(x):
    return x * 0.5 * (1.0 + lax.erf(x * 0.7071067811865476))


def _bn(x, g, b, m, v):
    s = g * lax.rsqrt(v + 1e-3)
    return x * s + (b - m * s)


def _prep_body(x_ref, g1, b1, m1, v1, W1, c1, g2, b2, m2, v2, W2, c2, o_ref):
    h = _bn(x_ref[...], g1[...], b1[...], m1[...], v1[...])
    h = _gelu(jnp.dot(h, W1[...], preferred_element_type=jnp.float32) + c1[...])
    h = _bn(h, g2[...], b2[...], m2[...], v2[...])
    m = _gelu(jnp.dot(h, W2[...], preferred_element_type=jnp.float32) + c2[...])
    o_ref[...] = jnp.concatenate(
        [m, jnp.ones((m.shape[0], _HP - m.shape[1]), jnp.float32)], axis=1)


def _prep_call(x, p1, p2, rb):
    n, d = x.shape
    h = p1[4].shape[1]
    grid = (n // rb,)

    def full(shape):
        return pl.BlockSpec(shape, lambda i: (0, 0))

    in_specs = [pl.BlockSpec((rb, d), lambda i: (i, 0))]
    for (g, b, m, v, W, c) in (p1, p2):
        din, dout = W.shape
        in_specs += [full((1, din))] * 4 + [full((din, dout)), full((1, dout))]
    args = [x]
    for (g, b, m, v, W, c) in (p1, p2):
        args += [g.reshape(1, -1), b.reshape(1, -1), m.reshape(1, -1),
                 v.reshape(1, -1), W, c.reshape(1, -1)]
    return pl.pallas_call(
        _prep_body,
        grid=grid,
        in_specs=in_specs,
        out_specs=pl.BlockSpec((rb, _HP), lambda i: (i, 0)),
        out_shape=jax.ShapeDtypeStruct((n, _HP), jnp.float32),
    )(*args)


_NB = 4
_LA = 2


def _sc_aggregate(msg, e3, w):
    n, h = msg.shape
    nw2, nchunk, ch = e3.shape
    epw = nchunk * ch
    assert nw2 == 2 * _NW and ch == _CH and h % _L == 0

    mesh = plsc.VectorSubcoreMesh(core_axis_name="c", subcore_axis_name="s")

    @functools.partial(
        pl.kernel,
        out_type=jax.ShapeDtypeStruct((_NC * n, 128), jnp.float32),
        mesh=mesh,
        scratch_types=[
            pltpu.VMEM((nchunk, _CH), jnp.int32),
            pltpu.VMEM((nchunk, _CH), jnp.int32),
            pltpu.VMEM((epw,), jnp.float32),
            pltpu.VMEM((_NB, _CH, h), jnp.float32),
            pltpu.VMEM_SHARED((n, h), jnp.float32),
            pltpu.SemaphoreType.DMA((_NB,)),
            pltpu.SemaphoreType.DMA((_NB,)),
        ],
        compiler_params=pltpu.CompilerParams(use_tc_tiling_on_sc=False,
                                             needs_layout_passes=False),
    )
    def body(msg_hbm, e_hbm, w_hbm, sums_out,
             sidx, didx, wv, ring, acc, gsem, ssem):
        c = lax.axis_index("c")
        s = lax.axis_index("s")
        wid = s * _NC + c

        pltpu.sync_copy(e_hbm.at[_NW + wid], sidx)
        pltpu.sync_copy(e_hbm.at[wid], didx)
        pltpu.sync_copy(w_hbm.at[pl.ds(wid * epw, epw)], wv)

        def wchk(i, acc0):
            v = wv[pl.ds(i * _L, _L)]
            nbad = plsc.all_reduce_population_count(v != 1.0)
            return acc0 + nbad[0]
        wdiff = lax.fori_loop(0, epw // _L, wchk, jnp.int32(0))
        allones = wdiff == 0

        def zrow(i, carry):
            for j in range(h // _L):
                ring[0, i, pl.ds(j * _L, _L)] = jnp.zeros((_L,), jnp.float32)
            return carry
        lax.fori_loop(0, _CH, zrow, 0)

        r0 = s * _ZB
        zb = ring.at[0]
        for j in range(_ZB // _CH):
            off = r0 + j * _CH

            @pl.when(off < n)
            def _():
                pltpu.async_copy(zb, acc.at[pl.ds(off, _CH)], gsem.at[0])
        for j in range(_ZB // _CH):
            off = r0 + j * _CH

            @pl.when(off < n)
            def _():
                pltpu.make_async_copy(zb, acc.at[pl.ds(off, _CH)], gsem.at[0]).wait()

        plsc.subcore_barrier()

        def fire_gather(k, b):
            pltpu.async_copy(msg_hbm.at[sidx.at[k]], ring.at[b], gsem.at[b])

        def drain_gather(b):
            pltpu.make_async_copy(msg_hbm.at[pl.ds(0, _CH)], ring.at[b],
                                  gsem.at[b]).wait()

        def fire_scatter(k, b):
            pltpu.async_copy(ring.at[b], acc.at[didx.at[k]], ssem.at[b],
                             add=True)

        def drain_scatter(b):
            pltpu.make_async_copy(msg_hbm.at[pl.ds(0, _CH)], ring.at[b],
                                  ssem.at[b]).wait()

        def scale(k, b):
            @pl.when(jnp.logical_not(allones))
            def _():
                def sgrp(g, cc):
                    wvec = wv[pl.ds(k * _CH + g * _L, _L)]
                    for j in range(_L):
                        wi = wvec[j]
                        i = g * _L + j
                        for q in range((h - _L) // _L):
                            ring[b, i, pl.ds(q * _L, _L)] = (
                                ring[b, i, pl.ds(q * _L, _L)] * wi)
                    return cc
                lax.fori_loop(0, _CH // _L, sgrp, 0)

        def step(k, b):
            kp = k + _LA
            bp = (b + _LA) % _NB

            @pl.when(kp < nchunk)
            def _():
                @pl.when(k >= _LA)
                def _():
                    drain_scatter(bp)
                fire_gather(kp, bp)
            drain_gather(b)
            scale(k, b)
            fire_scatter(k, b)

        for b in range(_LA):
            fire_gather(jnp.int32(b), b)

        nmain = (nchunk // _NB) * _NB

        def ring_step(g, carry):
            for b in range(_NB):
                step(g * _NB + b, b)
            return carry
        lax.fori_loop(0, nmain // _NB, ring_step, 0)

        for k in range(nmain, nchunk):
            step(jnp.int32(k), k % _NB)
        for k in range(nchunk - _NB, nchunk):
            drain_scatter(k % _NB)

        plsc.subcore_barrier()

        obase = c * n
        for j in range(_ZB // _CH):
            off = r0 + j * _CH
            b = j % _NB

            @pl.when(off < n)
            def _():
                if j >= _NB:
                    pltpu.make_async_copy(
                        ring.at[b],
                        sums_out.at[pl.ds(obase + off - _NB * _CH, _CH),
                                    pl.ds(0, h)],
                        gsem.at[b]).wait()
                pltpu.sync_copy(acc.at[pl.ds(off, _CH)], ring.at[b])
                pltpu.async_copy(
                    ring.at[b],
                    sums_out.at[pl.ds(obase + off, _CH), pl.ds(0, h)],
                    gsem.at[b])
        for j in range(_ZB // _CH):
            off = r0 + j * _CH
            b = j % _NB
            fired = off < n
            if j + _NB < _ZB // _CH:
                drained = (r0 + (j + _NB) * _CH) < n
                cond = jnp.logical_and(fired, jnp.logical_not(drained))
            else:
                cond = fired

            @pl.when(cond)
            def _():
                pltpu.make_async_copy(
                    ring.at[b],
                    sums_out.at[pl.ds(obase + off, _CH), pl.ds(0, h)],
                    gsem.at[b]).wait()

    return body(msg, e3, w)


def _upd_body(x_ref, sp_ref,
              g3x, b3x, m3x, v3x, g3a, b3a, m3a, v3a, W3x, W3a, c3,
              g4, b4, m4, v4, W4, c4, o_ref):
    s48 = sp_ref[0, :, : _HP] + sp_ref[1, :, : _HP]
    sums = s48[:, : _HP - _L]
    cnt = s48[:, _HP - _L: _HP - _L + 1]
    agg = sums / jnp.maximum(cnt, 1.0)
    hx = _bn(x_ref[...], g3x[...], b3x[...], m3x[...], v3x[...])
    ha = _bn(agg, g3a[...], b3a[...], m3a[...], v3a[...])
    t = _gelu(jnp.dot(hx, W3x[...], preferred_element_type=jnp.float32)
              + jnp.dot(ha, W3a[...], preferred_element_type=jnp.float32)
              + c3[...])
    t = _bn(t, g4[...], b4[...], m4[...], v4[...])
    o_ref[...] = _gelu(jnp.dot(t, W4[...], preferred_element_type=jnp.float32) + c4[...])


def _upd_call(x, sums_p, u1, u2, rb):
    n, d = x.shape
    h = _HP - _L
    grid = (n // rb,)
    g3, b3, m3, v3, W3, c3 = u1

    def full(shape):
        return pl.BlockSpec(shape, lambda i: tuple(0 for _ in shape))

    in_specs = [
        pl.BlockSpec((rb, d), lambda i: (i, 0)),
        pl.BlockSpec((_NC, rb, 128), lambda i: (0, i, 0)),
    ]
    in_specs += [full((1, d))] * 4 + [full((1, h))] * 4
    in_specs += [full((d, h)), full((h, h)), full((1, h))]
    in_specs += [full((1, h))] * 4 + [full((h, h)), full((1, h))]

    args = [x, sums_p,
            g3[:d].reshape(1, d), b3[:d].reshape(1, d),
            m3[:d].reshape(1, d), v3[:d].reshape(1, d),
            g3[d:].reshape(1, h), b3[d:].reshape(1, h),
            m3[d:].reshape(1, h), v3[d:].reshape(1, h),
            W3[:d], W3[d:], c3.reshape(1, h)]
    g4, b4, m4, v4, W4, c4 = u2
    args += [g4.reshape(1, h), b4.reshape(1, h), m4.reshape(1, h),
             v4.reshape(1, h), W4, c4.reshape(1, h)]
    return pl.pallas_call(
        _upd_body,
        grid=grid,
        in_specs=in_specs,
        out_specs=pl.BlockSpec((rb, h), lambda i: (i, 0)),
        out_shape=jax.ShapeDtypeStruct((n, h), jnp.float32),
    )(*args)


def kernel(node_representations, edges, edge_weights,
           p1_g, p1_b, p1_m, p1_v, p1_W, p1_c,
           p2_g, p2_b, p2_m, p2_v, p2_W, p2_c,
           u1_g, u1_b, u1_m, u1_v, u1_W, u1_c,
           u2_g, u2_b, u2_m, u2_v, u2_W, u2_c):
    x = node_representations.astype(jnp.float32)
    n, d = x.shape
    e = edges.shape[1]
    nchunk = e // (_NW * _CH)
    e3 = edges.astype(jnp.int32).reshape(2 * _NW, nchunk, _CH)
    w = edge_weights.astype(jnp.float32)

    msg = _prep_call(x, (p1_g, p1_b, p1_m, p1_v, p1_W, p1_c),
                     (p2_g, p2_b, p2_m, p2_v, p2_W, p2_c), rb=2000)
    sums2 = _sc_aggregate(msg, e3, w)
    sums_p = sums2.reshape(_NC, n, 128)
    return _upd_call(x, sums_p,
                     (u1_g, u1_b, u1_m, u1_v, u1_W, u1_c),
                     (u2_g, u2_b, u2_m, u2_v, u2_W, u2_c), rb=2000)


# speedup vs baseline: 23.4123x; 1.0084x over previous
import functools

import jax
import jax.numpy as jnp
from jax import lax
from jax.experimental import pallas as pl
from jax.experimental.pallas import tpu as pltpu
from jax.experimental.pallas import tpu_sc as plsc

_NC = 2
_NS = 16
_L = 16
_NW = _NC * _NS
_CH = 80
_ZB = 640
_HP = 48


def _gelu(x):
    return x * 0.5 * (1.0 + lax.erf(x * 0.7071067811865476))


def _bn(x, g, b, m, v):
    s = g * lax.rsqrt(v + 1e-3)
    return x * s + (b - m * s)


def _prep_body(x_ref, g1, b1, m1, v1, W1, c1, g2, b2, m2, v2, W2, c2, o_ref):
    h = _bn(x_ref[...], g1[...], b1[...], m1[...], v1[...])
    h = _gelu(jnp.dot(h, W1[...], preferred_element_type=jnp.float32) + c1[...])
    h = _bn(h, g2[...], b2[...], m2[...], v2[...])
    m = _gelu(jnp.dot(h, W2[...], preferred_element_type=jnp.float32) + c2[...])
    o_ref[...] = jnp.concatenate(
        [m, jnp.ones((m.shape[0], _HP - m.shape[1]), jnp.float32)], axis=1)


def _prep_call(x, p1, p2, rb):
    n, d = x.shape
    h = p1[4].shape[1]
    grid = (n // rb,)

    def full(shape):
        return pl.BlockSpec(shape, lambda i: (0, 0))

    in_specs = [pl.BlockSpec((rb, d), lambda i: (i, 0))]
    for (g, b, m, v, W, c) in (p1, p2):
        din, dout = W.shape
        in_specs += [full((1, din))] * 4 + [full((din, dout)), full((1, dout))]
    args = [x]
    for (g, b, m, v, W, c) in (p1, p2):
        args += [g.reshape(1, -1), b.reshape(1, -1), m.reshape(1, -1),
                 v.reshape(1, -1), W, c.reshape(1, -1)]
    return pl.pallas_call(
        _prep_body,
        grid=grid,
        in_specs=in_specs,
        out_specs=pl.BlockSpec((rb, _HP), lambda i: (i, 0)),
        out_shape=jax.ShapeDtypeStruct((n, _HP), jnp.float32),
    )(*args)


_NB = 4
_LA = 2


def _sc_aggregate(msg, e3, w):
    n, h = msg.shape
    nw2, nchunk, ch = e3.shape
    epw = nchunk * ch
    assert nw2 == 2 * _NW and ch == _CH and h % _L == 0

    mesh = plsc.VectorSubcoreMesh(core_axis_name="c", subcore_axis_name="s")

    @functools.partial(
        pl.kernel,
        out_type=jax.ShapeDtypeStruct((_NC * n, 128), jnp.float32),
        mesh=mesh,
        scratch_types=[
            pltpu.VMEM((nchunk, _CH), jnp.int32),
            pltpu.VMEM((nchunk, _CH), jnp.int32),
            pltpu.VMEM((epw,), jnp.float32),
            pltpu.VMEM((_NB, _CH, h), jnp.float32),
            pltpu.VMEM_SHARED((n, h), jnp.float32),
            pltpu.VMEM_SHARED((n, h), jnp.float32),
            pltpu.SemaphoreType.DMA((_NB,)),
            pltpu.SemaphoreType.DMA((_NB,)),
        ],
        compiler_params=pltpu.CompilerParams(use_tc_tiling_on_sc=False,
                                             needs_layout_passes=False),
    )
    def body(msg_hbm, e_hbm, w_hbm, sums_out,
             sidx, didx, wv, ring, msg_sh, acc, gsem, ssem):
        c = lax.axis_index("c")
        s = lax.axis_index("s")
        wid = s * _NC + c

        pltpu.sync_copy(e_hbm.at[_NW + wid], sidx)
        pltpu.sync_copy(e_hbm.at[wid], didx)
        pltpu.sync_copy(w_hbm.at[pl.ds(wid * epw, epw)], wv)

        def wchk(i, acc0):
            v = wv[pl.ds(i * _L, _L)]
            nbad = plsc.all_reduce_population_count(v != 1.0)
            return acc0 + nbad[0]
        wdiff = lax.fori_loop(0, epw // _L, wchk, jnp.int32(0))
        allones = wdiff == 0

        def zrow(i, carry):
            for j in range(h // _L):
                ring[0, i, pl.ds(j * _L, _L)] = jnp.zeros((_L,), jnp.float32)
            return carry
        lax.fori_loop(0, _CH, zrow, 0)

        r0 = s * _ZB
        zb = ring.at[0]
        for j in range(_ZB // _CH):
            off = r0 + j * _CH

            @pl.when(off < n)
            def _():
                pltpu.async_copy(zb, acc.at[pl.ds(off, _CH)], gsem.at[0])
                pltpu.async_copy(msg_hbm.at[pl.ds(off, _CH)],
                                 msg_sh.at[pl.ds(off, _CH)], gsem.at[1])
        for j in range(_ZB // _CH):
            off = r0 + j * _CH

            @pl.when(off < n)
            def _():
                pltpu.make_async_copy(zb, acc.at[pl.ds(off, _CH)], gsem.at[0]).wait()
                pltpu.make_async_copy(msg_hbm.at[pl.ds(off, _CH)],
                                      msg_sh.at[pl.ds(off, _CH)], gsem.at[1]).wait()

        plsc.subcore_barrier()

        def fire_gather(k, b):
            pltpu.async_copy(msg_sh.at[sidx.at[k]], ring.at[b], gsem.at[b])

        def drain_gather(b):
            pltpu.make_async_copy(msg_hbm.at[pl.ds(0, _CH)], ring.at[b],
                                  gsem.at[b]).wait()

        def fire_scatter(k, b):
            pltpu.async_copy(ring.at[b], acc.at[didx.at[k]], ssem.at[b],
                             add=True)

        def drain_scatter(b):
            pltpu.make_async_copy(msg_hbm.at[pl.ds(0, _CH)], ring.at[b],
                                  ssem.at[b]).wait()

        def scale(k, b):
            @pl.when(jnp.logical_not(allones))
            def _():
                def sgrp(g, cc):
                    wvec = wv[pl.ds(k * _CH + g * _L, _L)]
                    for j in range(_L):
                        wi = wvec[j]
                        i = g * _L + j
                        for q in range((h - _L) // _L):
                            ring[b, i, pl.ds(q * _L, _L)] = (
                                ring[b, i, pl.ds(q * _L, _L)] * wi)
                    return cc
                lax.fori_loop(0, _CH // _L, sgrp, 0)

        def step(k, b):
            kp = k + _LA
            bp = (b + _LA) % _NB

            @pl.when(kp < nchunk)
            def _():
                @pl.when(k >= _LA)
                def _():
                    drain_scatter(bp)
                fire_gather(kp, bp)
            drain_gather(b)
            scale(k, b)
            fire_scatter(k, b)

        for b in range(_LA):
            fire_gather(jnp.int32(b), b)

        nmain = (nchunk // _NB) * _NB

        def ring_step(g, carry):
            for b in range(_NB):
                step(g * _NB + b, b)
            return carry
        lax.fori_loop(0, nmain // _NB, ring_step, 0)

        for k in range(nmain, nchunk):
            step(jnp.int32(k), k % _NB)
        for k in range(nchunk - _NB, nchunk):
            drain_scatter(k % _NB)

        plsc.subcore_barrier()

        obase = c * n
        for j in range(_ZB // _CH):
            off = r0 + j * _CH
            b = j % _NB

            @pl.when(off < n)
            def _():
                if j >= _NB:
                    pltpu.make_async_copy(
                        ring.at[b],
                        sums_out.at[pl.ds(obase + off - _NB * _CH, _CH),
                                    pl.ds(0, h)],
                        gsem.at[b]).wait()
                pltpu.sync_copy(acc.at[pl.ds(off, _CH)], ring.at[b])
                pltpu.async_copy(
                    ring.at[b],
                    sums_out.at[pl.ds(obase + off, _CH), pl.ds(0, h)],
                    gsem.at[b])
        for j in range(_ZB // _CH):
            off = r0 + j * _CH
            b = j % _NB
            fired = off < n
            if j + _NB < _ZB // _CH:
                drained = (r0 + (j + _NB) * _CH) < n
                cond = jnp.logical_and(fired, jnp.logical_not(drained))
            else:
                cond = fired

            @pl.when(cond)
            def _():
                pltpu.make_async_copy(
                    ring.at[b],
                    sums_out.at[pl.ds(obase + off, _CH), pl.ds(0, h)],
                    gsem.at[b]).wait()

    return body(msg, e3, w)


def _upd_body(x_ref, sp_ref,
              g3x, b3x, m3x, v3x, g3a, b3a, m3a, v3a, W3x, W3a, c3,
              g4, b4, m4, v4, W4, c4, o_ref):
    s48 = sp_ref[0, :, : _HP] + sp_ref[1, :, : _HP]
    sums = s48[:, : _HP - _L]
    cnt = s48[:, _HP - _L: _HP - _L + 1]
    agg = sums / jnp.maximum(cnt, 1.0)
    hx = _bn(x_ref[...], g3x[...], b3x[...], m3x[...], v3x[...])
    ha = _bn(agg, g3a[...], b3a[...], m3a[...], v3a[...])
    t = _gelu(jnp.dot(hx, W3x[...], preferred_element_type=jnp.float32)
              + jnp.dot(ha, W3a[...], preferred_element_type=jnp.float32)
              + c3[...])
    t = _bn(t, g4[...], b4[...], m4[...], v4[...])
    o_ref[...] = _gelu(jnp.dot(t, W4[...], preferred_element_type=jnp.float32) + c4[...])


def _upd_call(x, sums_p, u1, u2, rb):
    n, d = x.shape
    h = _HP - _L
    grid = (n // rb,)
    g3, b3, m3, v3, W3, c3 = u1

    def full(shape):
        return pl.BlockSpec(shape, lambda i: tuple(0 for _ in shape))

    in_specs = [
        pl.BlockSpec((rb, d), lambda i: (i, 0)),
        pl.BlockSpec((_NC, rb, 128), lambda i: (0, i, 0)),
    ]
    in_specs += [full((1, d))] * 4 + [full((1, h))] * 4
    in_specs += [full((d, h)), full((h, h)), full((1, h))]
    in_specs += [full((1, h))] * 4 + [full((h, h)), full((1, h))]

    args = [x, sums_p,
            g3[:d].reshape(1, d), b3[:d].reshape(1, d),
            m3[:d].reshape(1, d), v3[:d].reshape(1, d),
            g3[d:].reshape(1, h), b3[d:].reshape(1, h),
            m3[d:].reshape(1, h), v3[d:].reshape(1, h),
            W3[:d], W3[d:], c3.reshape(1, h)]
    g4, b4, m4, v4, W4, c4 = u2
    args += [g4.reshape(1, h), b4.reshape(1, h), m4.reshape(1, h),
             v4.reshape(1, h), W4, c4.reshape(1, h)]
    return pl.pallas_call(
        _upd_body,
        grid=grid,
        in_specs=in_specs,
        out_specs=pl.BlockSpec((rb, h), lambda i: (i, 0)),
        out_shape=jax.ShapeDtypeStruct((n, h), jnp.float32),
    )(*args)


def kernel(node_representations, edges, edge_weights,
           p1_g, p1_b, p1_m, p1_v, p1_W, p1_c,
           p2_g, p2_b, p2_m, p2_v, p2_W, p2_c,
           u1_g, u1_b, u1_m, u1_v, u1_W, u1_c,
           u2_g, u2_b, u2_m, u2_v, u2_W, u2_c):
    x = node_representations.astype(jnp.float32)
    n, d = x.shape
    e = edges.shape[1]
    nchunk = e // (_NW * _CH)
    e3 = edges.astype(jnp.int32).reshape(2 * _NW, nchunk, _CH)
    w = edge_weights.astype(jnp.float32)

    msg = _prep_call(x, (p1_g, p1_b, p1_m, p1_v, p1_W, p1_c),
                     (p2_g, p2_b, p2_m, p2_v, p2_W, p2_c), rb=2000)
    sums2 = _sc_aggregate(msg, e3, w)
    sums_p = sums2.reshape(_NC, n, 128)
    return _upd_call(x, sums_p,
                     (u1_g, u1_b, u1_m, u1_v, u1_W, u1_c),
                     (u2_g, u2_b, u2_m, u2_v, u2_W, u2_c), rb=2000)


# speedup vs baseline: 25.1509x; 1.0743x over previous
import functools

import jax
import jax.numpy as jnp
from jax import lax
from jax.experimental import pallas as pl
from jax.experimental.pallas import tpu as pltpu
from jax.experimental.pallas import tpu_sc as plsc

_NC = 2
_NS = 16
_L = 16
_NW = _NC * _NS
_CH = 80
_ZB = 640
_HP = 48


def _gelu(x):
    return x * 0.5 * (1.0 + lax.erf(x * 0.7071067811865476))


def _bn(x, g, b, m, v):
    s = g * lax.rsqrt(v + 1e-3)
    return x * s + (b - m * s)


def _prep_body(x_ref, g1, b1, m1, v1, W1, c1, g2, b2, m2, v2, W2, c2, o_ref):
    h = _bn(x_ref[...], g1[...], b1[...], m1[...], v1[...])
    h = _gelu(jnp.dot(h, W1[...], preferred_element_type=jnp.float32) + c1[...])
    h = _bn(h, g2[...], b2[...], m2[...], v2[...])
    m = _gelu(jnp.dot(h, W2[...], preferred_element_type=jnp.float32) + c2[...])
    o_ref[...] = jnp.concatenate(
        [m, jnp.ones((m.shape[0], _HP - m.shape[1]), jnp.float32)], axis=1)


def _prep_call(x, p1, p2, rb):
    n, d = x.shape
    h = p1[4].shape[1]
    grid = (n // rb,)

    def full(shape):
        return pl.BlockSpec(shape, lambda i: (0, 0))

    in_specs = [pl.BlockSpec((rb, d), lambda i: (i, 0))]
    for (g, b, m, v, W, c) in (p1, p2):
        din, dout = W.shape
        in_specs += [full((1, din))] * 4 + [full((din, dout)), full((1, dout))]
    args = [x]
    for (g, b, m, v, W, c) in (p1, p2):
        args += [g.reshape(1, -1), b.reshape(1, -1), m.reshape(1, -1),
                 v.reshape(1, -1), W, c.reshape(1, -1)]
    return pl.pallas_call(
        _prep_body,
        grid=grid,
        in_specs=in_specs,
        out_specs=pl.BlockSpec((rb, _HP), lambda i: (i, 0)),
        out_shape=jax.ShapeDtypeStruct((n, _HP), jnp.float32),
    )(*args)


_NB = 4
_LA = 2


def _sc_aggregate(msg, e3, w):
    n, h = msg.shape
    nw2, nchunk, ch = e3.shape
    epw = nchunk * ch
    assert nw2 == 2 * _NW and ch == _CH and h % _L == 0

    mesh = plsc.VectorSubcoreMesh(core_axis_name="c", subcore_axis_name="s")

    @functools.partial(
        pl.kernel,
        out_type=jax.ShapeDtypeStruct((_NC * n, 128), jnp.float32),
        mesh=mesh,
        scratch_types=[
            pltpu.VMEM((nchunk, _CH), jnp.int32),
            pltpu.VMEM((nchunk, _CH), jnp.int32),
            pltpu.VMEM((epw,), jnp.float32),
            pltpu.VMEM((_NB, _CH, h), jnp.float32),
            pltpu.VMEM_SHARED((n, h), jnp.float32),
            pltpu.VMEM_SHARED((n, h), jnp.float32),
            pltpu.SemaphoreType.DMA((_NB,)),
            pltpu.SemaphoreType.DMA((_NB,)),
        ],
        compiler_params=pltpu.CompilerParams(use_tc_tiling_on_sc=False,
                                             needs_layout_passes=False),
    )
    def body(msg_hbm, e_hbm, w_hbm, sums_out,
             sidx, didx, wv, ring, msg_sh, acc, gsem, ssem):
        c = lax.axis_index("c")
        s = lax.axis_index("s")
        wid = s * _NC + c

        pltpu.sync_copy(e_hbm.at[_NW + wid], sidx)
        pltpu.sync_copy(e_hbm.at[wid], didx)
        pltpu.sync_copy(w_hbm.at[pl.ds(wid * epw, epw)], wv)

        def wchk(i, acc0):
            v = wv[pl.ds(i * _L, _L)]
            nbad = plsc.all_reduce_population_count(v != 1.0)
            return acc0 + nbad[0]
        wdiff = lax.fori_loop(0, epw // _L, wchk, jnp.int32(0))
        allones = wdiff == 0

        def zrow(i, carry):
            for j in range(h // _L):
                ring[0, i, pl.ds(j * _L, _L)] = jnp.zeros((_L,), jnp.float32)
            return carry
        lax.fori_loop(0, _CH, zrow, 0)

        r0 = s * _ZB
        zb = ring.at[0]
        for j in range(_ZB // _CH):
            off = r0 + j * _CH

            @pl.when(off < n)
            def _():
                pltpu.async_copy(zb, acc.at[pl.ds(off, _CH)], gsem.at[0])
                pltpu.async_copy(msg_hbm.at[pl.ds(off, _CH)],
                                 msg_sh.at[pl.ds(off, _CH)], gsem.at[1])
        for j in range(_ZB // _CH):
            off = r0 + j * _CH

            @pl.when(off < n)
            def _():
                pltpu.make_async_copy(zb, acc.at[pl.ds(off, _CH)], gsem.at[0]).wait()
                pltpu.make_async_copy(msg_hbm.at[pl.ds(off, _CH)],
                                      msg_sh.at[pl.ds(off, _CH)], gsem.at[1]).wait()

        plsc.subcore_barrier()

        def fire_gather(k, b):
            pltpu.async_copy(msg_sh.at[sidx.at[k]], ring.at[b], gsem.at[b])

        def drain_gather(b):
            pltpu.make_async_copy(msg_hbm.at[pl.ds(0, _CH)], ring.at[b],
                                  gsem.at[b]).wait()

        def fire_scatter(k, b):
            pltpu.async_copy(ring.at[b], acc.at[didx.at[k]], ssem.at[b],
                             add=True)

        def drain_scatter(b):
            pltpu.make_async_copy(msg_hbm.at[pl.ds(0, _CH)], ring.at[b],
                                  ssem.at[b]).wait()

        def scale(k, b):
            @pl.when(jnp.logical_not(allones))
            def _():
                def sgrp(g, cc):
                    wvec = wv[pl.ds(k * _CH + g * _L, _L)]
                    for j in range(_L):
                        wi = wvec[j]
                        i = g * _L + j
                        for q in range((h - _L) // _L):
                            ring[b, i, pl.ds(q * _L, _L)] = (
                                ring[b, i, pl.ds(q * _L, _L)] * wi)
                    return cc
                lax.fori_loop(0, _CH // _L, sgrp, 0)

        def step(k, b):
            kp = k + _LA
            bp = (b + _LA) % _NB

            @pl.when(kp < nchunk)
            def _():
                @pl.when(k >= _LA)
                def _():
                    drain_scatter(bp)
                fire_gather(kp, bp)
            drain_gather(b)
            scale(k, b)
            fire_scatter(k, b)

        for b in range(_LA):
            fire_gather(jnp.int32(b), b)

        nmain = (nchunk // _NB) * _NB

        def ring_step(g, carry):
            for b in range(_NB):
                step(g * _NB + b, b)
            return carry
        lax.fori_loop(0, nmain // _NB, ring_step, 0)

        for k in range(nmain, nchunk):
            step(jnp.int32(k), k % _NB)
        for k in range(nchunk - _NB, nchunk):
            drain_scatter(k % _NB)

        plsc.subcore_barrier()

        obase = c * n
        for j in range(_ZB // _CH):
            off = r0 + j * _CH
            b = j % _NB

            @pl.when(off < n)
            def _():
                if j >= _NB:
                    pltpu.make_async_copy(
                        ring.at[b],
                        sums_out.at[pl.ds(obase + off - _NB * _CH, _CH),
                                    pl.ds(0, h)],
                        gsem.at[b]).wait()
                pltpu.sync_copy(acc.at[pl.ds(off, _CH)], ring.at[b])
                pltpu.async_copy(
                    ring.at[b],
                    sums_out.at[pl.ds(obase + off, _CH), pl.ds(0, h)],
                    gsem.at[b])
        for j in range(_ZB // _CH):
            off = r0 + j * _CH
            b = j % _NB
            fired = off < n
            if j + _NB < _ZB // _CH:
                drained = (r0 + (j + _NB) * _CH) < n
                cond = jnp.logical_and(fired, jnp.logical_not(drained))
            else:
                cond = fired

            @pl.when(cond)
            def _():
                pltpu.make_async_copy(
                    ring.at[b],
                    sums_out.at[pl.ds(obase + off, _CH), pl.ds(0, h)],
                    gsem.at[b]).wait()

    return body(msg, e3, w)


def _upd_body(x_ref, sp_ref,
              g3x, b3x, m3x, v3x, g3a, b3a, m3a, v3a, W3x, W3a, c3,
              g4, b4, m4, v4, W4, c4, o_ref):
    s48 = sp_ref[0, :, : _HP] + sp_ref[1, :, : _HP]
    sums = s48[:, : _HP - _L]
    cnt = s48[:, _HP - _L: _HP - _L + 1]
    agg = sums / jnp.maximum(cnt, 1.0)
    hx = _bn(x_ref[...], g3x[...], b3x[...], m3x[...], v3x[...])
    ha = _bn(agg, g3a[...], b3a[...], m3a[...], v3a[...])
    t = _gelu(jnp.dot(hx, W3x[...], preferred_element_type=jnp.float32)
              + jnp.dot(ha, W3a[...], preferred_element_type=jnp.float32)
              + c3[...])
    t = _bn(t, g4[...], b4[...], m4[...], v4[...])
    ot = lax.dot_general(W4[...], t, (((0,), (1,)), ((), ())),
                         preferred_element_type=jnp.float32)
    o_ref[...] = _gelu(ot + c4[...])


def _upd_call(x, sums_p, u1, u2, rb):
    n, d = x.shape
    h = _HP - _L
    grid = (n // rb,)
    g3, b3, m3, v3, W3, c3 = u1

    def full(shape):
        return pl.BlockSpec(shape, lambda i: tuple(0 for _ in shape))

    in_specs = [
        pl.BlockSpec((rb, d), lambda i: (i, 0)),
        pl.BlockSpec((_NC, rb, 128), lambda i: (0, i, 0)),
    ]
    in_specs += [full((1, d))] * 4 + [full((1, h))] * 4
    in_specs += [full((d, h)), full((h, h)), full((1, h))]
    in_specs += [full((1, h))] * 4 + [full((h, h)), full((h, 1))]

    args = [x, sums_p,
            g3[:d].reshape(1, d), b3[:d].reshape(1, d),
            m3[:d].reshape(1, d), v3[:d].reshape(1, d),
            g3[d:].reshape(1, h), b3[d:].reshape(1, h),
            m3[d:].reshape(1, h), v3[d:].reshape(1, h),
            W3[:d], W3[d:], c3.reshape(1, h)]
    g4, b4, m4, v4, W4, c4 = u2
    args += [g4.reshape(1, h), b4.reshape(1, h), m4.reshape(1, h),
             v4.reshape(1, h), W4, c4.reshape(h, 1)]
    return pl.pallas_call(
        _upd_body,
        grid=grid,
        in_specs=in_specs,
        out_specs=pl.BlockSpec((h, rb), lambda i: (0, i)),
        out_shape=jax.ShapeDtypeStruct((h, n), jnp.float32),
    )(*args)


def kernel(node_representations, edges, edge_weights,
           p1_g, p1_b, p1_m, p1_v, p1_W, p1_c,
           p2_g, p2_b, p2_m, p2_v, p2_W, p2_c,
           u1_g, u1_b, u1_m, u1_v, u1_W, u1_c,
           u2_g, u2_b, u2_m, u2_v, u2_W, u2_c):
    x = node_representations.astype(jnp.float32)
    n, d = x.shape
    e = edges.shape[1]
    nchunk = e // (_NW * _CH)
    e3 = edges.astype(jnp.int32).reshape(2 * _NW, nchunk, _CH)
    w = edge_weights.astype(jnp.float32)

    msg = _prep_call(x, (p1_g, p1_b, p1_m, p1_v, p1_W, p1_c),
                     (p2_g, p2_b, p2_m, p2_v, p2_W, p2_c), rb=2000)
    sums2 = _sc_aggregate(msg, e3, w)
    sums_p = sums2.reshape(_NC, n, 128)
    out_t = _upd_call(x, sums_p,
                      (u1_g, u1_b, u1_m, u1_v, u1_W, u1_c),
                      (u2_g, u2_b, u2_m, u2_v, u2_W, u2_c), rb=n)
    return out_t.T
